# Initial kernel scaffold; baseline (speedup 1.0000x reference)
import jax
import jax.numpy as jnp
from jax.experimental import pallas as pl


def kernel(u, neighs, features):
    raise NotImplementedError("write your pallas kernel here")


# speedup vs baseline: 6.8833x; 6.8833x over previous
import functools

import jax
import jax.numpy as jnp
from jax import lax
from jax.experimental import pallas as pl
from jax.experimental.pallas import tpu as pltpu
from jax.experimental.pallas import tpu_sc as plsc

_LANES = 128


@functools.lru_cache(maxsize=None)
def _build_hist_kernel(rw, n_pad, nc, ns):
    mesh = plsc.VectorSubcoreMesh(core_axis_name="c", subcore_axis_name="s",
                                  num_cores=nc, num_subcores=ns)

    chunk = n_pad // ns

    @functools.partial(
        pl.kernel,
        out_type=jax.ShapeDtypeStruct((nc * n_pad,), jnp.float32),
        mesh=mesh,
        scratch_types=[
            pltpu.VMEM((rw, _LANES), jnp.int32),
            pltpu.VMEM((_LANES,), jnp.float32),
            pltpu.VMEM((chunk,), jnp.float32),
            pltpu.VMEM_SHARED((n_pad,), jnp.float32),
        ],
    )
    def hist_kernel(idx_hbm, out_hbm, idx_v, ones_v, hbuf_v, hist_sh):
        c = lax.axis_index("c")
        s = lax.axis_index("s")
        wid = s * nc + c

        for i in range(_LANES // 16):
            ones_v[pl.ds(i * 16, 16)] = jnp.ones((16,), jnp.float32)

        def zbody(i, carry):
            hbuf_v[pl.ds(i * 16, 16)] = jnp.zeros((16,), jnp.float32)
            return carry

        lax.fori_loop(0, chunk // 16, zbody, jnp.int32(0))
        pltpu.sync_copy(hbuf_v, hist_sh.at[pl.ds(s * chunk, chunk)])

        plsc.subcore_barrier()

        pltpu.sync_copy(idx_hbm.at[pl.ds(wid * rw, rw)], idx_v)

        def body(i, carry):
            pltpu.sync_copy(ones_v, hist_sh.at[idx_v.at[i]], add=True)
            return carry

        lax.fori_loop(0, rw, body, jnp.int32(0))

        plsc.subcore_barrier()

        pltpu.sync_copy(hist_sh.at[pl.ds(s * chunk, chunk)], hbuf_v)
        pltpu.sync_copy(hbuf_v,
                        out_hbm.at[pl.ds(c * n_pad + s * chunk, chunk)])

    return hist_kernel


def _matvec_body(nk, inv_e, w0_ref, w1_ref, f_ref, o_ref):
    k = pl.program_id(0)

    @pl.when(k == 0)
    def _():
        o_ref[...] = jnp.zeros_like(o_ref)

    w = w0_ref[...] + w1_ref[...]
    o_ref[...] += jnp.sum(f_ref[...] * w, axis=0, keepdims=True)

    @pl.when(k == nk - 1)
    def _():
        o_ref[...] = o_ref[...] * inv_e


def kernel(u, neighs, features):
    del u
    e = neighs.shape[0]
    n, d = features.shape

    info = plsc.get_sparse_core_info()
    nc, ns = info.num_cores, info.num_subcores
    nw = nc * ns
    rw = 8 * (-(-e // (_LANES * nw * 8)))
    rtot = rw * nw
    n_pad = ns * _LANES * (-(-(n + 8) // (ns * _LANES)))
    pad = rtot * _LANES - e
    idx2d = jnp.concatenate(
        [neighs.astype(jnp.int32),
         n + (jnp.arange(pad, dtype=jnp.int32) % 8)]).reshape(rtot, _LANES)

    hist = _build_hist_kernel(rw, n_pad, nc, ns)(idx2d)

    w0 = hist[:n].reshape(n, 1)
    w1 = hist[n_pad:n_pad + n].reshape(n, 1)

    kb = next(b for b in (5000, 4000, 2500, 2000, 1000, 500, 8, 1)
              if n % b == 0)
    nk = n // kb
    out = pl.pallas_call(
        functools.partial(_matvec_body, nk, 1.0 / e),
        grid=(nk,),
        in_specs=[
            pl.BlockSpec((kb, 1), lambda k: (k, 0)),
            pl.BlockSpec((kb, 1), lambda k: (k, 0)),
            pl.BlockSpec((kb, d), lambda k: (k, 0)),
        ],
        out_specs=pl.BlockSpec((1, d), lambda k: (0, 0)),
        out_shape=jax.ShapeDtypeStruct((1, d), jnp.float32),
    )(w0, w1, features)
    return out.reshape(d)


# speedup vs baseline: 20.1034x; 2.9206x over previous
import functools

import jax
import jax.numpy as jnp
from jax import lax
from jax.experimental import pallas as pl
from jax.experimental.pallas import tpu as pltpu
from jax.experimental.pallas import tpu_sc as plsc

_LANES = 128


@functools.lru_cache(maxsize=None)
def _build_hist_kernel(rw, n_pad, nc, ns):
    mesh = plsc.VectorSubcoreMesh(core_axis_name="c", subcore_axis_name="s",
                                  num_cores=nc, num_subcores=ns)

    chunk = n_pad // ns

    @functools.partial(
        pl.kernel,
        out_type=jax.ShapeDtypeStruct((nc * n_pad,), jnp.float32),
        mesh=mesh,
        scratch_types=[
            pltpu.VMEM((rw, _LANES), jnp.int32),
            pltpu.VMEM((_LANES,), jnp.float32),
            pltpu.VMEM((chunk,), jnp.float32),
            pltpu.VMEM_SHARED((n_pad,), jnp.float32),
        ],
    )
    def hist_kernel(idx_hbm, out_hbm, idx_v, ones_v, hbuf_v, hist_sh):
        c = lax.axis_index("c")
        s = lax.axis_index("s")
        wid = s * nc + c

        for i in range(_LANES // 16):
            ones_v[pl.ds(i * 16, 16)] = jnp.ones((16,), jnp.float32)

        def zbody(i, carry):
            hbuf_v[pl.ds(i * 16, 16)] = jnp.zeros((16,), jnp.float32)
            return carry

        lax.fori_loop(0, chunk // 16, zbody, jnp.int32(0))
        pltpu.sync_copy(hbuf_v, hist_sh.at[pl.ds(s * chunk, chunk)])

        plsc.subcore_barrier()

        pltpu.sync_copy(idx_hbm.at[pl.ds(wid * rw, rw)], idx_v)

        def body(i, carry):
            pltpu.sync_copy(ones_v, hist_sh.at[idx_v.at[i]], add=True)
            return carry

        lax.fori_loop(0, rw, body, jnp.int32(0))

        plsc.subcore_barrier()

        pltpu.sync_copy(hist_sh.at[pl.ds(s * chunk, chunk)], hbuf_v)
        pltpu.sync_copy(hbuf_v,
                        out_hbm.at[pl.ds(c * n_pad + s * chunk, chunk)])

    return hist_kernel


def _matvec_body(nk, kb, inv_e, w0_ref, w1_ref, f_ref, o_ref):
    k = pl.program_id(0)

    @pl.when(k == 0)
    def _():
        o_ref[...] = jnp.zeros_like(o_ref)

    w = (w0_ref[...] + w1_ref[...]).reshape(1, kb)
    o_ref[...] += jnp.dot(w, f_ref[...],
                          preferred_element_type=jnp.float32)

    @pl.when(k == nk - 1)
    def _():
        o_ref[...] = o_ref[...] * inv_e


def kernel(u, neighs, features):
    del u
    e = neighs.shape[0]
    n, d = features.shape

    info = plsc.get_sparse_core_info()
    nc, ns = info.num_cores, info.num_subcores
    nw = nc * ns
    rw = 8 * (-(-e // (_LANES * nw * 8)))
    rtot = rw * nw
    n_pad = ns * _LANES * (-(-(n + 8) // (ns * _LANES)))
    pad = rtot * _LANES - e
    idx2d = jnp.concatenate(
        [neighs.astype(jnp.int32),
         n + (jnp.arange(pad, dtype=jnp.int32) % 8)]).reshape(rtot, _LANES)

    hist = _build_hist_kernel(rw, n_pad, nc, ns)(idx2d)

    kb = next(b for b in (10000, 5000, 4000, 2500, 2000, 1000, 500, 8, 1)
              if n % b == 0)
    nk = n // kb
    w0 = hist[:n].reshape(nk, 1, kb)
    w1 = hist[n_pad:n_pad + n].reshape(nk, 1, kb)
    out = pl.pallas_call(
        functools.partial(_matvec_body, nk, kb, 1.0 / e),
        grid=(nk,),
        in_specs=[
            pl.BlockSpec((1, 1, kb), lambda k: (k, 0, 0)),
            pl.BlockSpec((1, 1, kb), lambda k: (k, 0, 0)),
            pl.BlockSpec((kb, d), lambda k: (k, 0)),
        ],
        out_specs=pl.BlockSpec((1, d), lambda k: (0, 0)),
        out_shape=jax.ShapeDtypeStruct((1, d), jnp.float32),
    )(w0, w1, features)
    return out.reshape(d)


# speedup vs baseline: 20.1638x; 1.0030x over previous
import functools

import jax
import jax.numpy as jnp
from jax import lax
from jax.experimental import pallas as pl
from jax.experimental.pallas import tpu as pltpu
from jax.experimental.pallas import tpu_sc as plsc

_LANES = 128


@functools.lru_cache(maxsize=None)
def _build_hist_kernel(rw, n_pad, nc, ns):
    mesh = plsc.VectorSubcoreMesh(core_axis_name="c", subcore_axis_name="s",
                                  num_cores=nc, num_subcores=ns)

    chunk = n_pad // ns

    ew = rw * _LANES

    @functools.partial(
        pl.kernel,
        out_type=jax.ShapeDtypeStruct((nc * n_pad,), jnp.float32),
        mesh=mesh,
        scratch_types=[
            pltpu.VMEM((ew,), jnp.int32),
            pltpu.VMEM((ew,), jnp.float32),
            pltpu.VMEM((chunk,), jnp.float32),
            pltpu.VMEM_SHARED((n_pad,), jnp.float32),
        ],
    )
    def hist_kernel(idx_hbm, out_hbm, idx_v, ones_v, hbuf_v, hist_sh):
        c = lax.axis_index("c")
        s = lax.axis_index("s")
        wid = s * nc + c

        def obody(i, carry):
            ones_v[pl.ds(i * 16, 16)] = jnp.ones((16,), jnp.float32)
            return carry

        lax.fori_loop(0, ew // 16, obody, jnp.int32(0))

        def zbody(i, carry):
            hbuf_v[pl.ds(i * 16, 16)] = jnp.zeros((16,), jnp.float32)
            return carry

        lax.fori_loop(0, chunk // 16, zbody, jnp.int32(0))
        pltpu.sync_copy(hbuf_v, hist_sh.at[pl.ds(s * chunk, chunk)])

        plsc.subcore_barrier()

        pltpu.sync_copy(idx_hbm.at[pl.ds(wid * ew, ew)], idx_v)

        pltpu.sync_copy(ones_v, hist_sh.at[idx_v], add=True)

        plsc.subcore_barrier()

        pltpu.sync_copy(hist_sh.at[pl.ds(s * chunk, chunk)], hbuf_v)
        pltpu.sync_copy(hbuf_v,
                        out_hbm.at[pl.ds(c * n_pad + s * chunk, chunk)])

    return hist_kernel


def _matvec_body(nk, kb, inv_e, w0_ref, w1_ref, f_ref, o_ref):
    k = pl.program_id(0)

    @pl.when(k == 0)
    def _():
        o_ref[...] = jnp.zeros_like(o_ref)

    w = (w0_ref[...] + w1_ref[...]).reshape(1, kb)
    o_ref[...] += jnp.dot(w, f_ref[...],
                          preferred_element_type=jnp.float32)

    @pl.when(k == nk - 1)
    def _():
        o_ref[...] = o_ref[...] * inv_e


def kernel(u, neighs, features):
    del u
    e = neighs.shape[0]
    n, d = features.shape

    info = plsc.get_sparse_core_info()
    nc, ns = info.num_cores, info.num_subcores
    nw = nc * ns
    rw = 8 * (-(-e // (_LANES * nw * 8)))
    rtot = rw * nw
    n_pad = ns * _LANES * (-(-(n + 8) // (ns * _LANES)))
    pad = rtot * _LANES - e
    idx1d = jnp.concatenate(
        [neighs.astype(jnp.int32),
         n + (jnp.arange(pad, dtype=jnp.int32) % 8)])

    hist = _build_hist_kernel(rw, n_pad, nc, ns)(idx1d)

    kb = next(b for b in (10000, 5000, 4000, 2500, 2000, 1000, 500, 8, 1)
              if n % b == 0)
    nk = n // kb
    w0 = hist[:n].reshape(nk, 1, kb)
    w1 = hist[n_pad:n_pad + n].reshape(nk, 1, kb)
    out = pl.pallas_call(
        functools.partial(_matvec_body, nk, kb, 1.0 / e),
        grid=(nk,),
        in_specs=[
            pl.BlockSpec((1, 1, kb), lambda k: (k, 0, 0)),
            pl.BlockSpec((1, 1, kb), lambda k: (k, 0, 0)),
            pl.BlockSpec((kb, d), lambda k: (k, 0)),
        ],
        out_specs=pl.BlockSpec((1, d), lambda k: (0, 0)),
        out_shape=jax.ShapeDtypeStruct((1, d), jnp.float32),
    )(w0, w1, features)
    return out.reshape(d)


# speedup vs baseline: 22.0649x; 1.0943x over previous
import functools

import jax
import jax.numpy as jnp
from jax import lax
from jax.experimental import pallas as pl
from jax.experimental.pallas import tpu as pltpu
from jax.experimental.pallas import tpu_sc as plsc


@functools.lru_cache(maxsize=None)
def _build_hist_kernel(e, n_pad, nc, ns):
    mesh = plsc.VectorSubcoreMesh(core_axis_name="c", subcore_axis_name="s",
                                  num_cores=nc, num_subcores=ns)
    nw = nc * ns
    chunk = n_pad // ns
    ew = 8 * (e // (nw * 8))
    tail = e - nw * ew
    ew16 = -16 * (-ew // 16)
    assert tail % 16 == 0 and chunk % 16 == 0

    scratch = [
        pltpu.VMEM((ew,), jnp.int32),
        pltpu.VMEM((ew16,), jnp.float32),
        pltpu.VMEM((chunk,), jnp.float32),
        pltpu.VMEM_SHARED((n_pad,), jnp.float32),
    ]
    if tail:
        scratch.append(pltpu.VMEM((tail,), jnp.int32))

    @functools.partial(
        pl.kernel,
        out_type=jax.ShapeDtypeStruct((nc * n_pad,), jnp.float32),
        mesh=mesh,
        scratch_types=scratch,
    )
    def hist_kernel(idx_hbm, out_hbm, idx_v, ones_v, hbuf_v, hist_sh,
                    *tail_v):
        c = lax.axis_index("c")
        s = lax.axis_index("s")
        wid = s * nc + c

        def obody(i, carry):
            ones_v[pl.ds(i * 16, 16)] = jnp.ones((16,), jnp.float32)
            return carry

        lax.fori_loop(0, ew16 // 16, obody, jnp.int32(0))

        def zbody(i, carry):
            hbuf_v[pl.ds(i * 16, 16)] = jnp.zeros((16,), jnp.float32)
            return carry

        lax.fori_loop(0, chunk // 16, zbody, jnp.int32(0))
        pltpu.sync_copy(hbuf_v, hist_sh.at[pl.ds(s * chunk, chunk)])

        plsc.subcore_barrier()

        pltpu.sync_copy(idx_hbm.at[pl.ds(wid * ew, ew)], idx_v)
        pltpu.sync_copy(ones_v.at[pl.ds(0, ew)], hist_sh.at[idx_v],
                        add=True)

        if tail:
            @pl.when(wid == nw - 1)
            def _():
                pltpu.sync_copy(idx_hbm.at[pl.ds(nw * ew, tail)], tail_v[0])
                pltpu.sync_copy(ones_v.at[pl.ds(0, tail)],
                                hist_sh.at[tail_v[0]], add=True)

        plsc.subcore_barrier()

        pltpu.sync_copy(hist_sh.at[pl.ds(s * chunk, chunk)], hbuf_v)
        pltpu.sync_copy(hbuf_v,
                        out_hbm.at[pl.ds(c * n_pad + s * chunk, chunk)])

    return hist_kernel


def _matvec_body(nk, kb8, inv_e, w0_ref, w1_ref, f_ref, o_ref):
    k = pl.program_id(0)

    @pl.when(k == 0)
    def _():
        o_ref[...] = jnp.zeros_like(o_ref)

    w = (w0_ref[...] + w1_ref[...])[0]
    acc = o_ref[...]
    for r in range(8):
        acc += jnp.dot(w[r:r + 1], f_ref[pl.ds(r * kb8, kb8), :],
                       preferred_element_type=jnp.float32)
    o_ref[...] = acc

    @pl.when(k == nk - 1)
    def _():
        o_ref[...] = o_ref[...] * inv_e


def kernel(u, neighs, features):
    del u
    e = neighs.shape[0]
    n, d = features.shape

    info = plsc.get_sparse_core_info()
    nc, ns = info.num_cores, info.num_subcores
    n_pad = ns * 128 * (-(-n // (ns * 128)))
    hist = _build_hist_kernel(e, n_pad, nc, ns)(neighs.astype(jnp.int32))

    kb = next(b for b in (10000, 5000, 4000, 2500, 2000, 1000, 500, 8)
              if n % b == 0 and b % 8 == 0)
    nk = n // kb
    kb8 = kb // 8
    w0 = hist[:n].reshape(nk, 8, kb8)
    w1 = hist[n_pad:n_pad + n].reshape(nk, 8, kb8)
    out = pl.pallas_call(
        functools.partial(_matvec_body, nk, kb8, 1.0 / e),
        grid=(nk,),
        in_specs=[
            pl.BlockSpec((1, 8, kb8), lambda k: (k, 0, 0)),
            pl.BlockSpec((1, 8, kb8), lambda k: (k, 0, 0)),
            pl.BlockSpec((kb, d), lambda k: (k, 0)),
        ],
        out_specs=pl.BlockSpec((1, d), lambda k: (0, 0)),
        out_shape=jax.ShapeDtypeStruct((1, d), jnp.float32),
    )(w0, w1, features)
    return out.reshape(d)


# speedup vs baseline: 22.8533x; 1.0357x over previous
import functools

import jax
import jax.numpy as jnp
from jax import lax
from jax.experimental import pallas as pl
from jax.experimental.pallas import tpu as pltpu
from jax.experimental.pallas import tpu_sc as plsc


@functools.lru_cache(maxsize=None)
def _build_hist_kernel(e, n, n_pad, nc, ns):
    mesh = plsc.VectorSubcoreMesh(core_axis_name="c", subcore_axis_name="s",
                                  num_cores=nc, num_subcores=ns)
    nw = nc * ns
    chunk = n_pad // ns
    ew = 8 * (e // (nw * 8))
    tail = e - nw * ew
    ew16 = -16 * (-ew // 16)
    cut = n - (ns - 1) * chunk
    padlen = chunk - cut
    assert tail % 16 == 0 and chunk % 16 == 0
    assert 0 < cut <= chunk and cut % 8 == 0

    scratch = [
        pltpu.VMEM((ew,), jnp.int32),
        pltpu.VMEM((ew16,), jnp.float32),
        pltpu.VMEM((chunk,), jnp.float32),
        pltpu.VMEM_SHARED((n_pad,), jnp.float32),
        pltpu.SemaphoreType.DMA,
    ]
    if tail:
        scratch.append(pltpu.VMEM((tail,), jnp.int32))

    @functools.partial(
        pl.kernel,
        out_type=(jax.ShapeDtypeStruct((nc * n,), jnp.float32),
                  jax.ShapeDtypeStruct((nc * padlen,), jnp.float32)),
        mesh=mesh,
        scratch_types=scratch,
    )
    def hist_kernel(idx_hbm, out_hbm, pad_hbm, idx_v, ones_v, hbuf_v,
                    hist_sh, idx_sem, *tail_v):
        c = lax.axis_index("c")
        s = lax.axis_index("s")
        wid = s * nc + c

        idx_cp = pltpu.async_copy(idx_hbm.at[pl.ds(wid * ew, ew)], idx_v,
                                  idx_sem)

        def obody(i, carry):
            ones_v[pl.ds(i * 16, 16)] = jnp.ones((16,), jnp.float32)
            return carry

        lax.fori_loop(0, ew16 // 16, obody, jnp.int32(0))

        def zbody(i, carry):
            hbuf_v[pl.ds(i * 16, 16)] = jnp.zeros((16,), jnp.float32)
            return carry

        lax.fori_loop(0, chunk // 16, zbody, jnp.int32(0))
        pltpu.sync_copy(hbuf_v, hist_sh.at[pl.ds(s * chunk, chunk)])

        plsc.subcore_barrier()
        idx_cp.wait()

        pltpu.sync_copy(ones_v.at[pl.ds(0, ew)], hist_sh.at[idx_v],
                        add=True)

        if tail:
            @pl.when(wid == nw - 1)
            def _():
                pltpu.sync_copy(idx_hbm.at[pl.ds(nw * ew, tail)], tail_v[0])
                pltpu.sync_copy(ones_v.at[pl.ds(0, tail)],
                                hist_sh.at[tail_v[0]], add=True)

        plsc.subcore_barrier()

        pltpu.sync_copy(hist_sh.at[pl.ds(s * chunk, chunk)], hbuf_v)

        @pl.when(s < ns - 1)
        def _():
            pltpu.sync_copy(hbuf_v, out_hbm.at[pl.ds(c * n + s * chunk,
                                                     chunk)])

        @pl.when(s == ns - 1)
        def _():
            pltpu.sync_copy(hbuf_v.at[pl.ds(0, cut)],
                            out_hbm.at[pl.ds(c * n + (ns - 1) * chunk, cut)])
            if padlen:
                pltpu.sync_copy(hbuf_v.at[pl.ds(cut, padlen)],
                                pad_hbm.at[pl.ds(c * padlen, padlen)])

    return hist_kernel


def _matvec_body(nk, kb8, inv_e, w0_ref, w1_ref, f_ref, o_ref):
    k = pl.program_id(0)

    @pl.when(k == 0)
    def _():
        o_ref[...] = jnp.zeros_like(o_ref)

    w = w0_ref[0, 0] + w1_ref[0, 0]
    acc = o_ref[...]
    for r in range(8):
        acc += jnp.dot(w[r:r + 1], f_ref[pl.ds(r * kb8, kb8), :],
                       preferred_element_type=jnp.float32)
    o_ref[...] = acc

    @pl.when(k == nk - 1)
    def _():
        o_ref[...] = o_ref[...] * inv_e


def kernel(u, neighs, features):
    del u
    e = neighs.shape[0]
    n, d = features.shape

    info = plsc.get_sparse_core_info()
    nc, ns = info.num_cores, info.num_subcores
    n_pad = ns * 128 * (-(-n // (ns * 128)))
    hist, _ = _build_hist_kernel(e, n, n_pad, nc, ns)(
        neighs.astype(jnp.int32))

    kb = next(b for b in (10000, 5000, 4000, 2500, 2000, 1000, 500, 8)
              if n % b == 0 and b % 8 == 0)
    nk = n // kb
    kb8 = kb // 8
    w = hist.reshape(nc, nk, 8, kb8)
    out = pl.pallas_call(
        functools.partial(_matvec_body, nk, kb8, 1.0 / e),
        grid=(nk,),
        in_specs=[
            pl.BlockSpec((1, 1, 8, kb8), lambda k: (0, k, 0, 0)),
            pl.BlockSpec((1, 1, 8, kb8), lambda k: (1, k, 0, 0)),
            pl.BlockSpec((kb, d), lambda k: (k, 0)),
        ],
        out_specs=pl.BlockSpec((1, d), lambda k: (0, 0)),
        out_shape=jax.ShapeDtypeStruct((1, d), jnp.float32),
    )(w, w, features)
    return out.reshape(d)
